# Initial kernel scaffold; baseline (speedup 1.0000x reference)
import jax
import jax.numpy as jnp
from jax.experimental import pallas as pl


def kernel(x, edge_attr, edge_index, params):
    raise NotImplementedError("write your pallas kernel here")


# speedup vs baseline: 2.6745x; 2.6745x over previous
import functools

import jax
import jax.numpy as jnp
from jax import lax
from jax.experimental import pallas as pl
from jax.experimental.pallas import tpu as pltpu
from jax.experimental.pallas import tpu_sc as plsc

N_NODES = 10000
N_EDGES = 160000
NODE_IN = 128
EDGE_IN = 16
HID = 128
DEPTH = 5
NODE_OUT = 256

NC, NS = 2, 16
NW = NC * NS
N_PAD = 10240
C = 32
EPW = 5056
E_PAD = EPW * NW
ACCW = 128
ROWS_PER_TILE = N_PAD // NS

BN = 512
GRID_N = N_PAD // BN
GRID_E = E_PAD // BN


def _leaky(v):
    return jnp.where(v >= 0, v, 0.2 * v)


def _tc_input_body(x_ref, win_ref, bin_ref, wcat_ref, bcat_ref,
                   ai_ref, s_ref, d_ref, wn_ref):
    ai = _leaky(jnp.dot(x_ref[:], win_ref[:],
                        preferred_element_type=jnp.float32) + bin_ref[:])
    ai_ref[:] = ai
    y = jnp.dot(ai, wcat_ref[:], preferred_element_type=jnp.float32) + bcat_ref[:]
    s_ref[:] = y[:, 0:256]
    d_ref[:] = y[:, 256:512]
    wn_ref[:] = y[:, 512:640]


def _tc_input(x_p, win, bin_, wcat, bcat):
    return pl.pallas_call(
        _tc_input_body,
        grid=(GRID_N,),
        in_specs=[
            pl.BlockSpec((BN, NODE_IN), lambda i: (i, 0)),
            pl.BlockSpec((NODE_IN, HID), lambda i: (0, 0)),
            pl.BlockSpec((1, HID), lambda i: (0, 0)),
            pl.BlockSpec((HID, 640), lambda i: (0, 0)),
            pl.BlockSpec((1, 640), lambda i: (0, 0)),
        ],
        out_specs=[
            pl.BlockSpec((BN, HID), lambda i: (i, 0)),
            pl.BlockSpec((BN, 256), lambda i: (i, 0)),
            pl.BlockSpec((BN, 256), lambda i: (i, 0)),
            pl.BlockSpec((BN, HID), lambda i: (i, 0)),
        ],
        out_shape=[
            jax.ShapeDtypeStruct((N_PAD, HID), jnp.float32),
            jax.ShapeDtypeStruct((N_PAD, 256), jnp.float32),
            jax.ShapeDtypeStruct((N_PAD, 256), jnp.float32),
            jax.ShapeDtypeStruct((N_PAD, HID), jnp.float32),
        ],
    )(x_p, win, bin_, wcat, bcat)


def _agg_from_accs(accs, dens):
    s = accs[0] + accs[1]
    den = jnp.sum(dens, axis=0).reshape(-1, 1)
    return s / (den + 1e-16)


def _tc_mid_body(accs_ref, dens_ref, wnb_ref, ai_ref, wcat_ref, bcat_ref,
                 s_ref, d_ref, wn_ref):
    agg = _agg_from_accs(accs_ref[:], dens_ref[:])
    h = jax.nn.relu(_leaky(agg + wnb_ref[:]) + ai_ref[:])
    y = jnp.dot(h, wcat_ref[:], preferred_element_type=jnp.float32) + bcat_ref[:]
    s_ref[:] = y[:, 0:256]
    d_ref[:] = y[:, 256:512]
    wn_ref[:] = y[:, 512:640]


def _tc_mid(accs, dens, wnb, ai, wcat, bcat):
    return pl.pallas_call(
        _tc_mid_body,
        grid=(GRID_N,),
        in_specs=[
            pl.BlockSpec((NC, BN, ACCW), lambda i: (0, i, 0)),
            pl.BlockSpec((NW, BN), lambda i: (0, i)),
            pl.BlockSpec((BN, HID), lambda i: (i, 0)),
            pl.BlockSpec((BN, HID), lambda i: (i, 0)),
            pl.BlockSpec((HID, 640), lambda i: (0, 0)),
            pl.BlockSpec((1, 640), lambda i: (0, 0)),
        ],
        out_specs=[
            pl.BlockSpec((BN, 256), lambda i: (i, 0)),
            pl.BlockSpec((BN, 256), lambda i: (i, 0)),
            pl.BlockSpec((BN, HID), lambda i: (i, 0)),
        ],
        out_shape=[
            jax.ShapeDtypeStruct((N_PAD, 256), jnp.float32),
            jax.ShapeDtypeStruct((N_PAD, 256), jnp.float32),
            jax.ShapeDtypeStruct((N_PAD, HID), jnp.float32),
        ],
    )(accs, dens, wnb, ai, wcat, bcat)


def _tc_edge_body(ea_ref, we_ref, be_ref, out_ref):
    out_ref[:] = jnp.dot(ea_ref[:], we_ref[:],
                         preferred_element_type=jnp.float32) + be_ref[:]


def _tc_edge(ea_p, we, be):
    return pl.pallas_call(
        _tc_edge_body,
        grid=(GRID_E,),
        in_specs=[
            pl.BlockSpec((BN, EDGE_IN), lambda i: (i, 0)),
            pl.BlockSpec((EDGE_IN, 256), lambda i: (0, 0)),
            pl.BlockSpec((1, 256), lambda i: (0, 0)),
        ],
        out_specs=pl.BlockSpec((BN, 256), lambda i: (i, 0)),
        out_shape=jax.ShapeDtypeStruct((E_PAD, 256), jnp.float32),
    )(ea_p, we, be)


def _tc_final_body(accs_ref, dens_ref, wnb_ref, ai_ref, x_ref, wox_ref, woh_ref,
                   bout_ref, wp_ref, bp_ref, gf_ref, out_ref):
    i = pl.program_id(0)
    agg = _agg_from_accs(accs_ref[:], dens_ref[:])
    h5 = jax.nn.relu(_leaky(agg + wnb_ref[:]) + ai_ref[:])
    no = _leaky(jnp.dot(x_ref[:], wox_ref[:], preferred_element_type=jnp.float32)
                + jnp.dot(h5, woh_ref[:], preferred_element_type=jnp.float32)
                + bout_ref[:])
    rid = i * BN + lax.broadcasted_iota(jnp.int32, (BN, 1), 0)
    no = jnp.where(rid < N_NODES, no, 0.0)
    part = jnp.sum(no, axis=0, keepdims=True)

    @pl.when(i == 0)
    def _():
        gf_ref[:] = part

    @pl.when(i > 0)
    def _():
        gf_ref[:] = gf_ref[:] + part

    @pl.when(i == GRID_N - 1)
    def _():
        out_ref[:] = (jnp.sum(gf_ref[:] * wp_ref[:]) + bp_ref[0, 0]).reshape(1, 1)


def _tc_final(accs, dens, wnb, ai, x_p, wox, woh, bout, wp_r, bp_a):
    gf, out = pl.pallas_call(
        _tc_final_body,
        grid=(GRID_N,),
        in_specs=[
            pl.BlockSpec((NC, BN, ACCW), lambda i: (0, i, 0)),
            pl.BlockSpec((NW, BN), lambda i: (0, i)),
            pl.BlockSpec((BN, HID), lambda i: (i, 0)),
            pl.BlockSpec((BN, HID), lambda i: (i, 0)),
            pl.BlockSpec((BN, NODE_IN), lambda i: (i, 0)),
            pl.BlockSpec((NODE_IN, NODE_OUT), lambda i: (0, 0)),
            pl.BlockSpec((HID, NODE_OUT), lambda i: (0, 0)),
            pl.BlockSpec((1, NODE_OUT), lambda i: (0, 0)),
            pl.BlockSpec((1, NODE_OUT), lambda i: (0, 0)),
            pl.BlockSpec((1, 1), lambda i: (0, 0)),
        ],
        out_specs=[
            pl.BlockSpec((1, NODE_OUT), lambda i: (0, 0)),
            pl.BlockSpec((1, 1), lambda i: (0, 0)),
        ],
        out_shape=[
            jax.ShapeDtypeStruct((1, NODE_OUT), jnp.float32),
            jax.ShapeDtypeStruct((1, 1), jnp.float32),
        ],
    )(accs, dens, wnb, ai, x_p, wox, woh, bout, wp_r, bp_a)
    del gf
    return out


_SC_MESH = plsc.VectorSubcoreMesh(core_axis_name="c", subcore_axis_name="s",
                                  num_cores=NC, num_subcores=NS)


@functools.partial(
    pl.kernel,
    mesh=_SC_MESH,
    compiler_params=pltpu.CompilerParams(needs_layout_passes=False),
    out_type=[
        jax.ShapeDtypeStruct((NC, N_PAD, ACCW), jnp.float32),
        jax.ShapeDtypeStruct((NW, N_PAD), jnp.float32),
    ],
    scratch_types=[
        pltpu.VMEM((C,), jnp.int32),
        pltpu.VMEM((C,), jnp.int32),
        pltpu.VMEM((C, 256), jnp.float32),
        pltpu.VMEM((C, 256), jnp.float32),
        pltpu.VMEM((C, 256), jnp.float32),
        pltpu.VMEM((C, ACCW), jnp.float32),
        pltpu.VMEM((HID,), jnp.float32),
        pltpu.VMEM((16, 16), jnp.float32),
        pltpu.VMEM((N_PAD,), jnp.float32),
        pltpu.VMEM_SHARED((N_PAD, ACCW), jnp.float32),
        pltpu.SemaphoreType.DMA,
    ],
)
def _sc_layer(s_hbm, d_hbm, e_hbm, srcs_hbm, dsts_hbm, wdot_hbm,
              out_hbm, den_hbm,
              srcidx, dstidx, buf_s, buf_d, buf_e, outbuf, wdotv, pbuf,
              denomv, acc_sh, sem):
    cid = lax.axis_index("c")
    sid = lax.axis_index("s")
    wid = sid * NC + cid
    ebase = wid * EPW

    zero16 = jnp.zeros((16,), jnp.float32)

    def zero_row(r, _):
        for j in range(ACCW // 16):
            outbuf[r, pl.ds(16 * j, 16)] = zero16
        return 0

    lax.fori_loop(0, C, zero_row, 0)

    def zero_den(t, _):
        denomv[pl.ds(t * 16, 16)] = zero16
        return 0

    lax.fori_loop(0, N_PAD // 16, zero_den, 0)

    def zero_acc(t, _):
        pltpu.sync_copy(outbuf, acc_sh.at[pl.ds(sid * ROWS_PER_TILE + t * C, C)])
        return 0

    lax.fori_loop(0, ROWS_PER_TILE // C, zero_acc, 0)
    plsc.subcore_barrier()

    pltpu.sync_copy(wdot_hbm, wdotv)
    wv = [wdotv[pl.ds(16 * j, 16)] for j in range(8)]

    def chunk_body(k, _):
        base = ebase + k * C
        pltpu.sync_copy(srcs_hbm.at[pl.ds(base, C)], srcidx)
        pltpu.sync_copy(dsts_hbm.at[pl.ds(base, C)], dstidx)
        cp1 = pltpu.async_copy(s_hbm.at[srcidx], buf_s, sem)
        cp2 = pltpu.async_copy(d_hbm.at[dstidx], buf_d, sem)
        cp3 = pltpu.async_copy(e_hbm.at[pl.ds(base, C)], buf_e, sem)
        cp1.wait()
        cp2.wait()
        cp3.wait()

        def group_body(g, _):
            def attn_body(i, _):
                e = g * 16 + i
                part = zero16
                for j in range(8):
                    sl = pl.ds(16 * j, 16)
                    u = buf_s[e, sl] + buf_d[e, sl] + buf_e[e, sl]
                    part = part + wv[j] * _leaky(u)
                pbuf[i, :] = part
                return 0

            lax.fori_loop(0, 16, attn_body, 0)

            iot = lax.iota(jnp.int32, 16)
            lg = zero16
            for j in range(16):
                lg = lg + plsc.load_gather(
                    pbuf, [iot, jnp.full((16,), j, jnp.int32)])
            exg = jnp.exp(lg)

            dvec = dstidx[pl.ds(g * 16, 16)]
            for i in range(16):
                e = g * 16 + i
                ex_s = exg[i]
                exs = jnp.full((16,), ex_s)
                dl = dvec[i]
                b16 = (dl // 16) * 16
                lane = dl - b16
                seg = denomv[pl.ds(b16, 16)]
                denomv[pl.ds(b16, 16)] = seg + jnp.where(
                    iot == lane, ex_s, 0.0)
                for j in range(8):
                    sl = pl.ds(128 + 16 * j, 16)
                    mu = buf_s[e, sl] + buf_d[e, sl] + buf_e[e, sl]
                    outbuf[e, pl.ds(16 * j, 16)] = exs * _leaky(mu)
            return 0

        lax.fori_loop(0, C // 16, group_body, 0)
        pltpu.sync_copy(outbuf, acc_sh.at[dstidx], add=True)
        return 0

    lax.fori_loop(0, EPW // C, chunk_body, 0)
    plsc.subcore_barrier()

    def write_out(t, _):
        r0 = sid * ROWS_PER_TILE + t * C
        pltpu.sync_copy(acc_sh.at[pl.ds(r0, C)], out_hbm.at[cid, pl.ds(r0, C)])
        return 0

    lax.fori_loop(0, ROWS_PER_TILE // C, write_out, 0)
    pltpu.sync_copy(denomv, den_hbm.at[wid])


def _row(v):
    return v.reshape(1, -1)


def kernel(x, edge_attr, edge_index, params):
    src = edge_index[0]
    dst = edge_index[1]
    srcs_p = jnp.zeros((E_PAD,), jnp.int32).at[:N_EDGES].set(src)
    dsts_p = jnp.full((E_PAD,), N_PAD - 1, jnp.int32).at[:N_EDGES].set(dst)
    ea_p = jnp.zeros((E_PAD, EDGE_IN), jnp.float32).at[:N_EDGES].set(edge_attr)
    x_p = jnp.zeros((N_PAD, NODE_IN), jnp.float32).at[:N_NODES].set(x)

    lp = params['layers']

    def layer_mats(p):
        wcat = jnp.concatenate([p['attn_src'][0], p['msg_src'][0],
                                p['attn_dst'][0], p['msg_dst'][0],
                                p['wgt_n'][0]], axis=1)
        bcat = jnp.concatenate([jnp.zeros((512,), jnp.float32),
                                p['wgt_n'][1]]).reshape(1, 640)
        we = jnp.concatenate([p['attn_edg'][0], p['msg_edg'][0]], axis=1)
        be = jnp.concatenate([
            p['attn_src'][1] + p['attn_dst'][1] + p['attn_edg'][1],
            p['msg_src'][1] + p['msg_dst'][1] + p['msg_edg'][1],
        ]).reshape(1, 256)
        wdot = p['attn_dot'][0][:, 0]
        return wcat, bcat, we, be, wdot

    mats = [layer_mats(p) for p in lp]

    win, bin_ = params['atom_inp']
    ai, s_t, d_t, wnb = _tc_input(x_p, win, _row(bin_), mats[0][0], mats[0][1])

    accs = dens = None
    for l in range(DEPTH):
        wcat, bcat, we, be, wdot = mats[l]
        if l > 0:
            s_t, d_t, wnb = _tc_mid(accs, dens, wnb, ai, wcat, bcat)
        eproj = _tc_edge(ea_p, we, be)
        accs, dens = _sc_layer(s_t, d_t, eproj, srcs_p, dsts_p, wdot)

    wout, bout = params['atom_out']
    wp, bp = params['predict']
    out = _tc_final(accs, dens, wnb, ai, x_p,
                    wout[:NODE_IN], wout[NODE_IN:], _row(bout),
                    wp[:, 0].reshape(1, NODE_OUT), bp.reshape(1, 1))
    return out


# speedup vs baseline: 4.6502x; 1.7387x over previous
import functools

import jax
import jax.numpy as jnp
from jax import lax
from jax.experimental import pallas as pl
from jax.experimental.pallas import tpu as pltpu
from jax.experimental.pallas import tpu_sc as plsc

N_NODES = 10000
N_EDGES = 160000
NODE_IN = 128
EDGE_IN = 16
HID = 128
DEPTH = 5
NODE_OUT = 256

NC, NS = 2, 16
NW = NC * NS
N_PAD = 10240
C = 16
EPW = 5056
E_PAD = EPW * NW
ACCW = 128
ROWS_PER_TILE = N_PAD // NS

BN = 512
GRID_N = N_PAD // BN
GRID_E = E_PAD // BN


def _leaky(v):
    return jnp.maximum(v, 0.2 * v)


def _tc_input_body(x_ref, win_ref, bin_ref, wcat_ref, bcat_ref,
                   ai_ref, s_ref, d_ref, wn_ref):
    ai = _leaky(jnp.dot(x_ref[:], win_ref[:],
                        preferred_element_type=jnp.float32) + bin_ref[:])
    ai_ref[:] = ai
    y = jnp.dot(ai, wcat_ref[:], preferred_element_type=jnp.float32) + bcat_ref[:]
    s_ref[:] = y[:, 0:256]
    d_ref[:] = y[:, 256:512]
    wn_ref[:] = y[:, 512:640]


def _tc_input(x_p, win, bin_, wcat, bcat):
    return pl.pallas_call(
        _tc_input_body,
        grid=(GRID_N,),
        in_specs=[
            pl.BlockSpec((BN, NODE_IN), lambda i: (i, 0)),
            pl.BlockSpec((NODE_IN, HID), lambda i: (0, 0)),
            pl.BlockSpec((1, HID), lambda i: (0, 0)),
            pl.BlockSpec((HID, 640), lambda i: (0, 0)),
            pl.BlockSpec((1, 640), lambda i: (0, 0)),
        ],
        out_specs=[
            pl.BlockSpec((BN, HID), lambda i: (i, 0)),
            pl.BlockSpec((BN, 256), lambda i: (i, 0)),
            pl.BlockSpec((BN, 256), lambda i: (i, 0)),
            pl.BlockSpec((BN, HID), lambda i: (i, 0)),
        ],
        out_shape=[
            jax.ShapeDtypeStruct((N_PAD, HID), jnp.float32),
            jax.ShapeDtypeStruct((N_PAD, 256), jnp.float32),
            jax.ShapeDtypeStruct((N_PAD, 256), jnp.float32),
            jax.ShapeDtypeStruct((N_PAD, HID), jnp.float32),
        ],
    )(x_p, win, bin_, wcat, bcat)


def _agg_from_accs(accs, dens):
    s = accs[0] + accs[1]
    den = jnp.sum(dens, axis=0).reshape(-1, 1)
    return s / (den + 1e-16)


def _tc_mid_body(accs_ref, dens_ref, wnb_ref, ai_ref, wcat_ref, bcat_ref,
                 s_ref, d_ref, wn_ref):
    agg = _agg_from_accs(accs_ref[:], dens_ref[:])
    h = jax.nn.relu(_leaky(agg + wnb_ref[:]) + ai_ref[:])
    y = jnp.dot(h, wcat_ref[:], preferred_element_type=jnp.float32) + bcat_ref[:]
    s_ref[:] = y[:, 0:256]
    d_ref[:] = y[:, 256:512]
    wn_ref[:] = y[:, 512:640]


def _tc_mid(accs, dens, wnb, ai, wcat, bcat):
    return pl.pallas_call(
        _tc_mid_body,
        grid=(GRID_N,),
        in_specs=[
            pl.BlockSpec((NC, BN, ACCW), lambda i: (0, i, 0)),
            pl.BlockSpec((NW, BN), lambda i: (0, i)),
            pl.BlockSpec((BN, HID), lambda i: (i, 0)),
            pl.BlockSpec((BN, HID), lambda i: (i, 0)),
            pl.BlockSpec((HID, 640), lambda i: (0, 0)),
            pl.BlockSpec((1, 640), lambda i: (0, 0)),
        ],
        out_specs=[
            pl.BlockSpec((BN, 256), lambda i: (i, 0)),
            pl.BlockSpec((BN, 256), lambda i: (i, 0)),
            pl.BlockSpec((BN, HID), lambda i: (i, 0)),
        ],
        out_shape=[
            jax.ShapeDtypeStruct((N_PAD, 256), jnp.float32),
            jax.ShapeDtypeStruct((N_PAD, 256), jnp.float32),
            jax.ShapeDtypeStruct((N_PAD, HID), jnp.float32),
        ],
    )(accs, dens, wnb, ai, wcat, bcat)


def _tc_edge_body(ea_ref, we_ref, be_ref, out_ref):
    out_ref[:] = jnp.dot(ea_ref[:], we_ref[:],
                         preferred_element_type=jnp.float32) + be_ref[:]


def _tc_edge(ea_p, we, be):
    return pl.pallas_call(
        _tc_edge_body,
        grid=(GRID_E,),
        in_specs=[
            pl.BlockSpec((BN, EDGE_IN), lambda i: (i, 0)),
            pl.BlockSpec((EDGE_IN, 256), lambda i: (0, 0)),
            pl.BlockSpec((1, 256), lambda i: (0, 0)),
        ],
        out_specs=pl.BlockSpec((BN, 256), lambda i: (i, 0)),
        out_shape=jax.ShapeDtypeStruct((E_PAD, 256), jnp.float32),
    )(ea_p, we, be)


def _tc_final_body(accs_ref, dens_ref, wnb_ref, ai_ref, x_ref, wox_ref, woh_ref,
                   bout_ref, wp_ref, bp_ref, gf_ref, out_ref):
    i = pl.program_id(0)
    agg = _agg_from_accs(accs_ref[:], dens_ref[:])
    h5 = jax.nn.relu(_leaky(agg + wnb_ref[:]) + ai_ref[:])
    no = _leaky(jnp.dot(x_ref[:], wox_ref[:], preferred_element_type=jnp.float32)
                + jnp.dot(h5, woh_ref[:], preferred_element_type=jnp.float32)
                + bout_ref[:])
    rid = i * BN + lax.broadcasted_iota(jnp.int32, (BN, 1), 0)
    no = jnp.where(rid < N_NODES, no, 0.0)
    part = jnp.sum(no, axis=0, keepdims=True)

    @pl.when(i == 0)
    def _():
        gf_ref[:] = part

    @pl.when(i > 0)
    def _():
        gf_ref[:] = gf_ref[:] + part

    @pl.when(i == GRID_N - 1)
    def _():
        out_ref[:] = (jnp.sum(gf_ref[:] * wp_ref[:]) + bp_ref[0, 0]).reshape(1, 1)


def _tc_final(accs, dens, wnb, ai, x_p, wox, woh, bout, wp_r, bp_a):
    gf, out = pl.pallas_call(
        _tc_final_body,
        grid=(GRID_N,),
        in_specs=[
            pl.BlockSpec((NC, BN, ACCW), lambda i: (0, i, 0)),
            pl.BlockSpec((NW, BN), lambda i: (0, i)),
            pl.BlockSpec((BN, HID), lambda i: (i, 0)),
            pl.BlockSpec((BN, HID), lambda i: (i, 0)),
            pl.BlockSpec((BN, NODE_IN), lambda i: (i, 0)),
            pl.BlockSpec((NODE_IN, NODE_OUT), lambda i: (0, 0)),
            pl.BlockSpec((HID, NODE_OUT), lambda i: (0, 0)),
            pl.BlockSpec((1, NODE_OUT), lambda i: (0, 0)),
            pl.BlockSpec((1, NODE_OUT), lambda i: (0, 0)),
            pl.BlockSpec((1, 1), lambda i: (0, 0)),
        ],
        out_specs=[
            pl.BlockSpec((1, NODE_OUT), lambda i: (0, 0)),
            pl.BlockSpec((1, 1), lambda i: (0, 0)),
        ],
        out_shape=[
            jax.ShapeDtypeStruct((1, NODE_OUT), jnp.float32),
            jax.ShapeDtypeStruct((1, 1), jnp.float32),
        ],
    )(accs, dens, wnb, ai, x_p, wox, woh, bout, wp_r, bp_a)
    del gf
    return out


_SC_MESH = plsc.VectorSubcoreMesh(core_axis_name="c", subcore_axis_name="s",
                                  num_cores=NC, num_subcores=NS)


NCH = EPW // C


@functools.partial(
    pl.kernel,
    mesh=_SC_MESH,
    compiler_params=pltpu.CompilerParams(needs_layout_passes=False),
    out_type=[
        jax.ShapeDtypeStruct((NC, N_PAD, ACCW), jnp.float32),
        jax.ShapeDtypeStruct((NW, N_PAD), jnp.float32),
    ],
    scratch_types=[
        pltpu.VMEM((C,), jnp.int32),
        pltpu.VMEM((C,), jnp.int32),
        pltpu.VMEM((C,), jnp.int32),
        pltpu.VMEM((C,), jnp.int32),
        pltpu.VMEM((C, 256), jnp.float32),
        pltpu.VMEM((C, 256), jnp.float32),
        pltpu.VMEM((C, 256), jnp.float32),
        pltpu.VMEM((C, 256), jnp.float32),
        pltpu.VMEM((C, 256), jnp.float32),
        pltpu.VMEM((C, 256), jnp.float32),
        pltpu.VMEM((C, ACCW), jnp.float32),
        pltpu.VMEM((HID,), jnp.float32),
        pltpu.VMEM((16, 16), jnp.float32),
        pltpu.VMEM((16,), jnp.float32),
        pltpu.VMEM((N_PAD,), jnp.float32),
        pltpu.VMEM_SHARED((N_PAD, ACCW), jnp.float32),
        pltpu.SemaphoreType.DMA,
        pltpu.SemaphoreType.DMA,
        pltpu.SemaphoreType.DMA,
        pltpu.SemaphoreType.DMA,
    ],
)
def _sc_layer(s_hbm, d_hbm, e_hbm, srcs_hbm, dsts_hbm, wdot_hbm,
              out_hbm, den_hbm,
              sidx0, sidx1, didx0, didx1, buf_s0, buf_s1, buf_d0, buf_d1,
              buf_e0, buf_e1, outbuf, wdotv, pbuf, exbuf, denomv, acc_sh,
              sem0, sem1, isem0, isem1):
    cid = lax.axis_index("c")
    sid = lax.axis_index("s")
    wid = sid * NC + cid
    ebase = wid * EPW

    bufs = ((buf_s0, buf_d0, buf_e0, sem0), (buf_s1, buf_d1, buf_e1, sem1))
    idxs = ((sidx0, didx0, isem0), (sidx1, didx1, isem1))
    zero16 = jnp.zeros((16,), jnp.float32)
    iot = lax.iota(jnp.int32, 16)

    def zero_row(r, _):
        for j in range(ACCW // 16):
            outbuf[r, pl.ds(16 * j, 16)] = zero16
        return 0

    lax.fori_loop(0, C, zero_row, 0)

    def zero_den(t, _):
        denomv[pl.ds(t * 16, 16)] = zero16
        return 0

    lax.fori_loop(0, N_PAD // 16, zero_den, 0)

    def zero_acc(t, _):
        pltpu.sync_copy(outbuf, acc_sh.at[pl.ds(sid * ROWS_PER_TILE + t * C, C)])
        return 0

    lax.fori_loop(0, ROWS_PER_TILE // C, zero_acc, 0)
    plsc.subcore_barrier()

    pltpu.sync_copy(wdot_hbm, wdotv)
    wv = [wdotv[pl.ds(16 * j, 16)] for j in range(8)]

    def issue_idx(k, iset):
        si, di, isem = iset
        pltpu.async_copy(srcs_hbm.at[pl.ds(ebase + k * C, C)], si, isem)
        pltpu.async_copy(dsts_hbm.at[pl.ds(ebase + k * C, C)], di, isem)

    def drain_idx(k, iset):
        si, di, isem = iset
        pltpu.make_async_copy(srcs_hbm.at[pl.ds(ebase + k * C, C)], si, isem).wait()
        pltpu.make_async_copy(dsts_hbm.at[pl.ds(ebase + k * C, C)], di, isem).wait()

    def issue(k, bset, iset):
        bs, bd, be, sem = bset
        si, di, _ = iset
        pltpu.async_copy(s_hbm.at[si], bs, sem)
        pltpu.async_copy(d_hbm.at[di], bd, sem)
        pltpu.async_copy(e_hbm.at[pl.ds(ebase + k * C, C)], be, sem)

    def drain(k, bset, iset):
        bs, bd, be, sem = bset
        si, di, _ = iset
        pltpu.make_async_copy(s_hbm.at[si], bs, sem).wait()
        pltpu.make_async_copy(d_hbm.at[di], bd, sem).wait()
        pltpu.make_async_copy(e_hbm.at[pl.ds(ebase + k * C, C)], be, sem).wait()

    issue_idx(0, idxs[0])
    drain_idx(0, idxs[0])
    issue(0, bufs[0], idxs[0])
    issue_idx(1, idxs[1])

    def chunk_pair(k2, _):
        for b in range(2):
            k = k2 * 2 + b
            buf_s, buf_d, buf_e, _sem = bufs[b]
            _si, didx, _isem = idxs[b]

            @pl.when(k + 1 < NCH)
            def _():
                drain_idx(k + 1, idxs[1 - b])
                issue(k + 1, bufs[1 - b], idxs[1 - b])

            drain(k, bufs[b], idxs[b])

            def attn_body(i, _):
                part0 = zero16
                part1 = zero16
                for j in range(8):
                    sl = pl.ds(16 * j, 16)
                    u = buf_s[i, sl] + buf_d[i, sl] + buf_e[i, sl]
                    t = wv[j] * _leaky(u)
                    if j % 2 == 0:
                        part0 = part0 + t
                    else:
                        part1 = part1 + t
                pbuf[i, :] = part0 + part1
                return 0

            lax.fori_loop(0, 16, attn_body, 0)

            lg0 = zero16
            lg1 = zero16
            for j in range(16):
                t = plsc.load_gather(pbuf, [iot, jnp.full((16,), j, jnp.int32)])
                if j % 2 == 0:
                    lg0 = lg0 + t
                else:
                    lg1 = lg1 + t
            exbuf[:] = jnp.exp(lg0 + lg1)

            dvec = didx[:]
            for i in range(16):
                exs = plsc.load_gather(exbuf, [jnp.full((16,), i, jnp.int32)])
                dl = dvec[i]
                b16 = (dl // 16) * 16
                lane = dl - b16
                seg = denomv[pl.ds(b16, 16)]
                denomv[pl.ds(b16, 16)] = seg + jnp.where(iot == lane, exs, 0.0)
                for j in range(8):
                    sl = pl.ds(128 + 16 * j, 16)
                    mu = buf_s[i, sl] + buf_d[i, sl] + buf_e[i, sl]
                    outbuf[i, pl.ds(16 * j, 16)] = exs * _leaky(mu)

            pltpu.sync_copy(outbuf, acc_sh.at[didx], add=True)

            @pl.when(k + 2 < NCH)
            def _():
                issue_idx(k + 2, idxs[b])
        return 0

    lax.fori_loop(0, NCH // 2, chunk_pair, 0)
    plsc.subcore_barrier()

    def write_out(t, _):
        r0 = sid * ROWS_PER_TILE + t * C
        pltpu.sync_copy(acc_sh.at[pl.ds(r0, C)], out_hbm.at[cid, pl.ds(r0, C)])
        return 0

    lax.fori_loop(0, ROWS_PER_TILE // C, write_out, 0)
    pltpu.sync_copy(denomv, den_hbm.at[wid])


def _row(v):
    return v.reshape(1, -1)


def kernel(x, edge_attr, edge_index, params):
    src = edge_index[0]
    dst = edge_index[1]
    srcs_p = jnp.zeros((E_PAD,), jnp.int32).at[:N_EDGES].set(src)
    dsts_p = jnp.full((E_PAD,), N_PAD - 1, jnp.int32).at[:N_EDGES].set(dst)
    ea_p = jnp.zeros((E_PAD, EDGE_IN), jnp.float32).at[:N_EDGES].set(edge_attr)
    x_p = jnp.zeros((N_PAD, NODE_IN), jnp.float32).at[:N_NODES].set(x)

    lp = params['layers']

    def layer_mats(p):
        wcat = jnp.concatenate([p['attn_src'][0], p['msg_src'][0],
                                p['attn_dst'][0], p['msg_dst'][0],
                                p['wgt_n'][0]], axis=1)
        bcat = jnp.concatenate([jnp.zeros((512,), jnp.float32),
                                p['wgt_n'][1]]).reshape(1, 640)
        we = jnp.concatenate([p['attn_edg'][0], p['msg_edg'][0]], axis=1)
        be = jnp.concatenate([
            p['attn_src'][1] + p['attn_dst'][1] + p['attn_edg'][1],
            p['msg_src'][1] + p['msg_dst'][1] + p['msg_edg'][1],
        ]).reshape(1, 256)
        wdot = p['attn_dot'][0][:, 0]
        return wcat, bcat, we, be, wdot

    mats = [layer_mats(p) for p in lp]

    win, bin_ = params['atom_inp']
    ai, s_t, d_t, wnb = _tc_input(x_p, win, _row(bin_), mats[0][0], mats[0][1])

    accs = dens = None
    for l in range(DEPTH):
        wcat, bcat, we, be, wdot = mats[l]
        if l > 0:
            s_t, d_t, wnb = _tc_mid(accs, dens, wnb, ai, wcat, bcat)
        eproj = _tc_edge(ea_p, we, be)
        accs, dens = _sc_layer(s_t, d_t, eproj, srcs_p, dsts_p, wdot)

    wout, bout = params['atom_out']
    wp, bp = params['predict']
    out = _tc_final(accs, dens, wnb, ai, x_p,
                    wout[:NODE_IN], wout[NODE_IN:], _row(bout),
                    wp[:, 0].reshape(1, NODE_OUT), bp.reshape(1, 1))
    return out


# speedup vs baseline: 4.8677x; 1.0468x over previous
import functools

import jax
import jax.numpy as jnp
from jax import lax
from jax.experimental import pallas as pl
from jax.experimental.pallas import tpu as pltpu
from jax.experimental.pallas import tpu_sc as plsc

N_NODES = 10000
N_EDGES = 160000
NODE_IN = 128
EDGE_IN = 16
HID = 128
DEPTH = 5
NODE_OUT = 256

NC, NS = 2, 16
NW = NC * NS
N_PAD = 10240
C = 16
EPW = 5056
E_PAD = EPW * NW
ACCW = 128
ROWS_PER_TILE = N_PAD // NS

BN = 512
GRID_N = N_PAD // BN
GRID_E = E_PAD // BN


def _leaky(v):
    return jnp.maximum(v, 0.2 * v)


def _tc_input_body(x_ref, win_ref, bin_ref, wcat_ref, bcat_ref,
                   ai_ref, s_ref, d_ref, wn_ref):
    ai = _leaky(jnp.dot(x_ref[:], win_ref[:],
                        preferred_element_type=jnp.float32) + bin_ref[:])
    ai_ref[:] = ai
    y = jnp.dot(ai, wcat_ref[:], preferred_element_type=jnp.float32) + bcat_ref[:]
    s_ref[:] = y[:, 0:256]
    d_ref[:] = y[:, 256:512]
    wn_ref[:] = y[:, 512:640]


def _tc_input(x_p, win, bin_, wcat, bcat):
    return pl.pallas_call(
        _tc_input_body,
        grid=(GRID_N,),
        in_specs=[
            pl.BlockSpec((BN, NODE_IN), lambda i: (i, 0)),
            pl.BlockSpec((NODE_IN, HID), lambda i: (0, 0)),
            pl.BlockSpec((1, HID), lambda i: (0, 0)),
            pl.BlockSpec((HID, 640), lambda i: (0, 0)),
            pl.BlockSpec((1, 640), lambda i: (0, 0)),
        ],
        out_specs=[
            pl.BlockSpec((BN, HID), lambda i: (i, 0)),
            pl.BlockSpec((BN, 256), lambda i: (i, 0)),
            pl.BlockSpec((BN, 256), lambda i: (i, 0)),
            pl.BlockSpec((BN, HID), lambda i: (i, 0)),
        ],
        out_shape=[
            jax.ShapeDtypeStruct((N_PAD, HID), jnp.float32),
            jax.ShapeDtypeStruct((N_PAD, 256), jnp.float32),
            jax.ShapeDtypeStruct((N_PAD, 256), jnp.float32),
            jax.ShapeDtypeStruct((N_PAD, HID), jnp.float32),
        ],
    )(x_p, win, bin_, wcat, bcat)


def _agg_from_accs(accs, dens):
    s = accs[0] + accs[1]
    den = jnp.sum(dens, axis=0).reshape(-1, 1)
    return s / (den + 1e-16)


def _tc_mid_body(accs_ref, dens_ref, wnb_ref, ai_ref, wcat_ref, bcat_ref,
                 s_ref, d_ref, wn_ref):
    agg = _agg_from_accs(accs_ref[:], dens_ref[:])
    h = jax.nn.relu(_leaky(agg + wnb_ref[:]) + ai_ref[:])
    y = jnp.dot(h, wcat_ref[:], preferred_element_type=jnp.float32) + bcat_ref[:]
    s_ref[:] = y[:, 0:256]
    d_ref[:] = y[:, 256:512]
    wn_ref[:] = y[:, 512:640]


def _tc_mid(accs, dens, wnb, ai, wcat, bcat):
    return pl.pallas_call(
        _tc_mid_body,
        grid=(GRID_N,),
        in_specs=[
            pl.BlockSpec((NC, BN, ACCW), lambda i: (0, i, 0)),
            pl.BlockSpec((NW, BN), lambda i: (0, i)),
            pl.BlockSpec((BN, HID), lambda i: (i, 0)),
            pl.BlockSpec((BN, HID), lambda i: (i, 0)),
            pl.BlockSpec((HID, 640), lambda i: (0, 0)),
            pl.BlockSpec((1, 640), lambda i: (0, 0)),
        ],
        out_specs=[
            pl.BlockSpec((BN, 256), lambda i: (i, 0)),
            pl.BlockSpec((BN, 256), lambda i: (i, 0)),
            pl.BlockSpec((BN, HID), lambda i: (i, 0)),
        ],
        out_shape=[
            jax.ShapeDtypeStruct((N_PAD, 256), jnp.float32),
            jax.ShapeDtypeStruct((N_PAD, 256), jnp.float32),
            jax.ShapeDtypeStruct((N_PAD, HID), jnp.float32),
        ],
    )(accs, dens, wnb, ai, wcat, bcat)


def _tc_edge_body(ea_ref, we_ref, be_ref, out_ref):
    out_ref[:] = jnp.dot(ea_ref[:], we_ref[:],
                         preferred_element_type=jnp.float32) + be_ref[:]


def _tc_edge(ea_p, we, be):
    return pl.pallas_call(
        _tc_edge_body,
        grid=(GRID_E,),
        in_specs=[
            pl.BlockSpec((BN, EDGE_IN), lambda i: (i, 0)),
            pl.BlockSpec((EDGE_IN, 256), lambda i: (0, 0)),
            pl.BlockSpec((1, 256), lambda i: (0, 0)),
        ],
        out_specs=pl.BlockSpec((BN, 256), lambda i: (i, 0)),
        out_shape=jax.ShapeDtypeStruct((E_PAD, 256), jnp.float32),
    )(ea_p, we, be)


def _tc_final_body(accs_ref, dens_ref, wnb_ref, ai_ref, x_ref, wox_ref, woh_ref,
                   bout_ref, wp_ref, bp_ref, gf_ref, out_ref):
    i = pl.program_id(0)
    agg = _agg_from_accs(accs_ref[:], dens_ref[:])
    h5 = jax.nn.relu(_leaky(agg + wnb_ref[:]) + ai_ref[:])
    no = _leaky(jnp.dot(x_ref[:], wox_ref[:], preferred_element_type=jnp.float32)
                + jnp.dot(h5, woh_ref[:], preferred_element_type=jnp.float32)
                + bout_ref[:])
    rid = i * BN + lax.broadcasted_iota(jnp.int32, (BN, 1), 0)
    no = jnp.where(rid < N_NODES, no, 0.0)
    part = jnp.sum(no, axis=0, keepdims=True)

    @pl.when(i == 0)
    def _():
        gf_ref[:] = part

    @pl.when(i > 0)
    def _():
        gf_ref[:] = gf_ref[:] + part

    @pl.when(i == GRID_N - 1)
    def _():
        out_ref[:] = (jnp.sum(gf_ref[:] * wp_ref[:]) + bp_ref[0, 0]).reshape(1, 1)


def _tc_final(accs, dens, wnb, ai, x_p, wox, woh, bout, wp_r, bp_a):
    gf, out = pl.pallas_call(
        _tc_final_body,
        grid=(GRID_N,),
        in_specs=[
            pl.BlockSpec((NC, BN, ACCW), lambda i: (0, i, 0)),
            pl.BlockSpec((NW, BN), lambda i: (0, i)),
            pl.BlockSpec((BN, HID), lambda i: (i, 0)),
            pl.BlockSpec((BN, HID), lambda i: (i, 0)),
            pl.BlockSpec((BN, NODE_IN), lambda i: (i, 0)),
            pl.BlockSpec((NODE_IN, NODE_OUT), lambda i: (0, 0)),
            pl.BlockSpec((HID, NODE_OUT), lambda i: (0, 0)),
            pl.BlockSpec((1, NODE_OUT), lambda i: (0, 0)),
            pl.BlockSpec((1, NODE_OUT), lambda i: (0, 0)),
            pl.BlockSpec((1, 1), lambda i: (0, 0)),
        ],
        out_specs=[
            pl.BlockSpec((1, NODE_OUT), lambda i: (0, 0)),
            pl.BlockSpec((1, 1), lambda i: (0, 0)),
        ],
        out_shape=[
            jax.ShapeDtypeStruct((1, NODE_OUT), jnp.float32),
            jax.ShapeDtypeStruct((1, 1), jnp.float32),
        ],
    )(accs, dens, wnb, ai, x_p, wox, woh, bout, wp_r, bp_a)
    del gf
    return out


_SC_MESH = plsc.VectorSubcoreMesh(core_axis_name="c", subcore_axis_name="s",
                                  num_cores=NC, num_subcores=NS)


NCH = EPW // C


@functools.partial(
    pl.kernel,
    mesh=_SC_MESH,
    compiler_params=pltpu.CompilerParams(needs_layout_passes=False),
    out_type=[
        jax.ShapeDtypeStruct((NC, N_PAD, ACCW), jnp.float32),
        jax.ShapeDtypeStruct((NW, N_PAD), jnp.float32),
    ],
    scratch_types=[
        pltpu.VMEM((C,), jnp.int32),
        pltpu.VMEM((C,), jnp.int32),
        pltpu.VMEM((C,), jnp.int32),
        pltpu.VMEM((C,), jnp.int32),
        pltpu.VMEM((C, 256), jnp.float32),
        pltpu.VMEM((C, 256), jnp.float32),
        pltpu.VMEM((C, 256), jnp.float32),
        pltpu.VMEM((C, 256), jnp.float32),
        pltpu.VMEM((C, 256), jnp.float32),
        pltpu.VMEM((C, 256), jnp.float32),
        pltpu.VMEM((C, ACCW), jnp.float32),
        pltpu.VMEM((C, ACCW), jnp.float32),
        pltpu.VMEM((C,), jnp.int32),
        pltpu.VMEM((C,), jnp.int32),
        pltpu.VMEM((HID,), jnp.float32),
        pltpu.VMEM((16, 16), jnp.float32),
        pltpu.VMEM((16,), jnp.float32),
        pltpu.VMEM((N_PAD,), jnp.float32),
        pltpu.VMEM_SHARED((N_PAD, ACCW), jnp.float32),
        pltpu.SemaphoreType.DMA,
        pltpu.SemaphoreType.DMA,
        pltpu.SemaphoreType.DMA,
        pltpu.SemaphoreType.DMA,
        pltpu.SemaphoreType.DMA,
        pltpu.SemaphoreType.DMA,
    ],
)
def _sc_layer(s_hbm, d_hbm, e_hbm, srcs_hbm, dsts_hbm, wdot_hbm,
              out_hbm, den_hbm,
              sidx0, sidx1, didx0, didx1, buf_s0, buf_s1, buf_d0, buf_d1,
              buf_e0, buf_e1, outbuf0, outbuf1, scidx0, scidx1, wdotv, pbuf,
              exbuf, denomv, acc_sh,
              sem0, sem1, isem0, isem1, scsem0, scsem1):
    cid = lax.axis_index("c")
    sid = lax.axis_index("s")
    wid = sid * NC + cid
    ebase = wid * EPW

    bufs = ((buf_s0, buf_d0, buf_e0, sem0), (buf_s1, buf_d1, buf_e1, sem1))
    idxs = ((sidx0, didx0, isem0), (sidx1, didx1, isem1))
    outs = ((outbuf0, scidx0, scsem0), (outbuf1, scidx1, scsem1))
    zero16 = jnp.zeros((16,), jnp.float32)
    iot = lax.iota(jnp.int32, 16)

    def zero_row(r, _):
        for j in range(ACCW // 16):
            outbuf0[r, pl.ds(16 * j, 16)] = zero16
        return 0

    lax.fori_loop(0, C, zero_row, 0)

    def zero_den(t, _):
        denomv[pl.ds(t * 16, 16)] = zero16
        return 0

    lax.fori_loop(0, N_PAD // 16, zero_den, 0)

    def zero_acc(t, _):
        pltpu.sync_copy(outbuf0, acc_sh.at[pl.ds(sid * ROWS_PER_TILE + t * C, C)])
        return 0

    lax.fori_loop(0, ROWS_PER_TILE // C, zero_acc, 0)
    plsc.subcore_barrier()

    pltpu.sync_copy(wdot_hbm, wdotv)
    wv = [wdotv[pl.ds(16 * j, 16)] for j in range(8)]

    def issue_idx(k, iset):
        si, di, isem = iset
        pltpu.async_copy(srcs_hbm.at[pl.ds(ebase + k * C, C)], si, isem)
        pltpu.async_copy(dsts_hbm.at[pl.ds(ebase + k * C, C)], di, isem)

    def drain_idx(k, iset):
        si, di, isem = iset
        pltpu.make_async_copy(srcs_hbm.at[pl.ds(ebase + k * C, C)], si, isem).wait()
        pltpu.make_async_copy(dsts_hbm.at[pl.ds(ebase + k * C, C)], di, isem).wait()

    def issue(k, bset, iset):
        bs, bd, be, sem = bset
        si, di, _ = iset
        pltpu.async_copy(s_hbm.at[si], bs, sem)
        pltpu.async_copy(d_hbm.at[di], bd, sem)
        pltpu.async_copy(e_hbm.at[pl.ds(ebase + k * C, C)], be, sem)

    def drain(k, bset, iset):
        bs, bd, be, sem = bset
        si, di, _ = iset
        pltpu.make_async_copy(s_hbm.at[si], bs, sem).wait()
        pltpu.make_async_copy(d_hbm.at[di], bd, sem).wait()
        pltpu.make_async_copy(e_hbm.at[pl.ds(ebase + k * C, C)], be, sem).wait()

    issue_idx(0, idxs[0])
    drain_idx(0, idxs[0])
    issue(0, bufs[0], idxs[0])
    issue_idx(1, idxs[1])

    def chunk_pair(k2, _):
        for b in range(2):
            k = k2 * 2 + b
            buf_s, buf_d, buf_e, _sem = bufs[b]
            _si, didx, _isem = idxs[b]
            outbuf, scidx, scsem = outs[b]

            @pl.when(k + 1 < NCH)
            def _():
                drain_idx(k + 1, idxs[1 - b])
                issue(k + 1, bufs[1 - b], idxs[1 - b])

            drain(k, bufs[b], idxs[b])

            @pl.when(k >= 2)
            def _():
                pltpu.make_async_copy(
                    outbuf, acc_sh.at[scidx], scsem).wait()

            def attn_body(i, _):
                part0 = zero16
                part1 = zero16
                for j in range(8):
                    sl = pl.ds(16 * j, 16)
                    u = buf_s[i, sl] + buf_d[i, sl] + buf_e[i, sl]
                    t = wv[j] * _leaky(u)
                    if j % 2 == 0:
                        part0 = part0 + t
                    else:
                        part1 = part1 + t
                pbuf[i, :] = part0 + part1
                return 0

            lax.fori_loop(0, 16, attn_body, 0)

            lg0 = zero16
            lg1 = zero16
            for j in range(16):
                t = plsc.load_gather(pbuf, [iot, jnp.full((16,), j, jnp.int32)])
                if j % 2 == 0:
                    lg0 = lg0 + t
                else:
                    lg1 = lg1 + t
            exbuf[:] = jnp.exp(lg0 + lg1)

            dvec = didx[:]
            for i in range(16):
                exs = plsc.load_gather(exbuf, [jnp.full((16,), i, jnp.int32)])
                dl = dvec[i]
                b16 = (dl // 16) * 16
                lane = dl - b16
                seg = denomv[pl.ds(b16, 16)]
                denomv[pl.ds(b16, 16)] = seg + jnp.where(iot == lane, exs, 0.0)
                for j in range(8):
                    sl = pl.ds(128 + 16 * j, 16)
                    mu = buf_s[i, sl] + buf_d[i, sl] + buf_e[i, sl]
                    outbuf[i, pl.ds(16 * j, 16)] = exs * _leaky(mu)

            scidx[:] = dvec
            pltpu.async_copy(outbuf, acc_sh.at[scidx], scsem, add=True)

            @pl.when(k + 2 < NCH)
            def _():
                issue_idx(k + 2, idxs[b])
        return 0

    lax.fori_loop(0, NCH // 2, chunk_pair, 0)
    for b in range(2):
        outbuf, scidx, scsem = outs[b]
        pltpu.make_async_copy(outbuf, acc_sh.at[scidx], scsem).wait()
    plsc.subcore_barrier()

    def write_out(t, _):
        r0 = sid * ROWS_PER_TILE + t * C
        pltpu.sync_copy(acc_sh.at[pl.ds(r0, C)], out_hbm.at[cid, pl.ds(r0, C)])
        return 0

    lax.fori_loop(0, ROWS_PER_TILE // C, write_out, 0)
    pltpu.sync_copy(denomv, den_hbm.at[wid])


def _row(v):
    return v.reshape(1, -1)


def kernel(x, edge_attr, edge_index, params):
    src = edge_index[0]
    dst = edge_index[1]
    srcs_p = jnp.zeros((E_PAD,), jnp.int32).at[:N_EDGES].set(src)
    dsts_p = jnp.full((E_PAD,), N_PAD - 1, jnp.int32).at[:N_EDGES].set(dst)
    ea_p = jnp.zeros((E_PAD, EDGE_IN), jnp.float32).at[:N_EDGES].set(edge_attr)
    x_p = jnp.zeros((N_PAD, NODE_IN), jnp.float32).at[:N_NODES].set(x)

    lp = params['layers']

    def layer_mats(p):
        wcat = jnp.concatenate([p['attn_src'][0], p['msg_src'][0],
                                p['attn_dst'][0], p['msg_dst'][0],
                                p['wgt_n'][0]], axis=1)
        bcat = jnp.concatenate([jnp.zeros((512,), jnp.float32),
                                p['wgt_n'][1]]).reshape(1, 640)
        we = jnp.concatenate([p['attn_edg'][0], p['msg_edg'][0]], axis=1)
        be = jnp.concatenate([
            p['attn_src'][1] + p['attn_dst'][1] + p['attn_edg'][1],
            p['msg_src'][1] + p['msg_dst'][1] + p['msg_edg'][1],
        ]).reshape(1, 256)
        wdot = p['attn_dot'][0][:, 0]
        return wcat, bcat, we, be, wdot

    mats = [layer_mats(p) for p in lp]

    win, bin_ = params['atom_inp']
    ai, s_t, d_t, wnb = _tc_input(x_p, win, _row(bin_), mats[0][0], mats[0][1])

    accs = dens = None
    for l in range(DEPTH):
        wcat, bcat, we, be, wdot = mats[l]
        if l > 0:
            s_t, d_t, wnb = _tc_mid(accs, dens, wnb, ai, wcat, bcat)
        eproj = _tc_edge(ea_p, we, be)
        accs, dens = _sc_layer(s_t, d_t, eproj, srcs_p, dsts_p, wdot)

    wout, bout = params['atom_out']
    wp, bp = params['predict']
    out = _tc_final(accs, dens, wnb, ai, x_p,
                    wout[:NODE_IN], wout[NODE_IN:], _row(bout),
                    wp[:, 0].reshape(1, NODE_OUT), bp.reshape(1, 1))
    return out
